# baseline (speedup 1.0000x reference)
import functools

import jax
import jax.numpy as jnp
from jax import lax
from jax.experimental import pallas as pl
from jax.experimental.pallas import tpu as pltpu
from jax.experimental.pallas import tpu_sc as plsc

D = 64
B = 4096
L = 200
LT = L // 8
BT = B // 128

NC = 2
NS = 16
NW = NC * NS
CHUNK = 128
NCHUNK = L
NBUF = 4
NOUTER = NCHUNK // NBUF
SCALE = 8.0


@functools.partial(
    pl.kernel,
    out_type=jax.ShapeDtypeStruct((L, D // 8, BT, 8, 128), jnp.float32),
    mesh=plsc.VectorSubcoreMesh(core_axis_name="c", subcore_axis_name="s"),
    scratch_types=[
        pltpu.VMEM((LT, 8, 128), jnp.int32),
        [pltpu.VMEM((CHUNK, D), jnp.float32) for _ in range(NBUF)],
        [pltpu.VMEM((D // 8, 8, 129), jnp.float32) for _ in range(NBUF)],
        [pltpu.SemaphoreType.DMA for _ in range(NBUF)],
        [pltpu.SemaphoreType.DMA for _ in range(NBUF)],
    ],
    compiler_params=pltpu.CompilerParams(
        use_tc_tiling_on_sc=False,
        needs_layout_passes=False,
        disable_bounds_checks=True,
    ),
)
def _embed(x_hbm, table_hbm, out_hbm, idx_v, gbufs, obufs, gsems, osems):
    wid = lax.axis_index("s") * NC + lax.axis_index("c")
    pltpu.sync_copy(x_hbm.at[:, wid], idx_v)

    def gather_desc(c, buf, sem):
        lt = c // 8
        li = lax.rem(c, 8)
        return pltpu.make_async_copy(
            table_hbm.at[idx_v.at[lt, li]], buf, sem
        )

    def start_gather(c, buf, sem):
        gather_desc(c, buf, sem).start()

    def wait_gather(c, buf, sem):
        gather_desc(c, buf, sem).wait()

    def start_out(c, buf, sem):
        pltpu.async_copy(
            buf.at[:, :, pl.ds(0, 128)], out_hbm.at[c, :, wid], sem
        )

    def wait_out(buf, sem):
        pltpu.make_async_copy(
            buf.at[:, :, pl.ds(0, 128)], out_hbm.at[0, :, wid], sem
        ).wait()

    lane = lax.iota(jnp.int32, 16)
    dhis = [(jnp.int32(d0) + lane) // 8 for d0 in range(0, D, 16)]
    dlos = [lax.rem(jnp.int32(d0) + lane, 8) for d0 in range(0, D, 16)]

    ROWS_PER_IT = 8

    def scale_t(gbuf, obuf):
        def body(it, carry):
            b0 = it * ROWS_PER_IT
            bvecs = [
                jnp.full((16,), 0, jnp.int32) + (b0 + r)
                for r in range(ROWS_PER_IT)
            ]
            vals = [
                gbuf[b0 + r, pl.ds(k * 16, 16)] * SCALE
                for r in range(ROWS_PER_IT)
                for k in range(D // 16)
            ]
            for r in range(ROWS_PER_IT):
                for k in range(D // 16):
                    plsc.store_scatter(
                        obuf,
                        [dhis[k], dlos[k], bvecs[r]],
                        vals[r * (D // 16) + k],
                    )
            return carry

        lax.fori_loop(0, CHUNK // ROWS_PER_IT, body, 0)

    for b in range(NBUF - 1):
        start_gather(b, gbufs[b], gsems[b])

    def outer(p, carry):
        for b in range(NBUF):
            g = p * NBUF + b
            nb = (b + NBUF - 1) % NBUF
            nxt = g + NBUF - 1

            @pl.when(nxt < NCHUNK)
            def _issue():
                if b == 0:

                    @pl.when(p > 0)
                    def _():
                        wait_out(obufs[nb], osems[nb])

                else:
                    wait_out(obufs[nb], osems[nb])
                start_gather(nxt, gbufs[nb], gsems[nb])

            wait_gather(g, gbufs[b], gsems[b])
            scale_t(gbufs[b], obufs[b])
            start_out(g, obufs[b], osems[b])
        return carry

    lax.fori_loop(0, NOUTER, outer, 0)

    for b in range(NBUF):
        wait_out(obufs[b], osems[b])


def kernel(x, table):
    xv = jnp.swapaxes(x.astype(jnp.int32).T.reshape(LT, 8, BT, 128), 1, 2)
    out5 = _embed(xv, table)
    return out5.transpose(2, 4, 0, 1, 3).reshape(B, L, D)


# speedup vs baseline: 1.0059x; 1.0059x over previous
import functools

import jax
import jax.numpy as jnp
from jax import lax
from jax.experimental import pallas as pl
from jax.experimental.pallas import tpu as pltpu
from jax.experimental.pallas import tpu_sc as plsc

D = 64
B = 4096
L = 200
LT = L // 8
BT = B // 128

NC = 2
NS = 16
NW = NC * NS
CHUNK = 128
NCHUNK = L
NBUF = 4
NOUTER = NCHUNK // NBUF
SCALE = 8.0


@functools.partial(
    pl.kernel,
    out_type=jax.ShapeDtypeStruct((L, D // 8, BT, 8, 128), jnp.float32),
    mesh=plsc.VectorSubcoreMesh(core_axis_name="c", subcore_axis_name="s"),
    scratch_types=[
        pltpu.VMEM((LT, 8, 128), jnp.int32),
        [pltpu.VMEM((CHUNK, D), jnp.float32) for _ in range(NBUF)],
        [pltpu.VMEM((D, 129), jnp.float32) for _ in range(NBUF)],
        [pltpu.SemaphoreType.DMA for _ in range(NBUF)],
        [pltpu.SemaphoreType.DMA for _ in range(NBUF)],
    ],
    compiler_params=pltpu.CompilerParams(
        use_tc_tiling_on_sc=False,
        needs_layout_passes=False,
        disable_bounds_checks=True,
    ),
)
def _embed(x_hbm, table_hbm, out_hbm, idx_v, gbufs, obufs, gsems, osems):
    wid = lax.axis_index("s") * NC + lax.axis_index("c")
    pltpu.sync_copy(x_hbm.at[:, wid], idx_v)

    def gather_desc(c, buf, sem):
        lt = c // 8
        li = lax.rem(c, 8)
        return pltpu.make_async_copy(
            table_hbm.at[idx_v.at[lt, li]], buf, sem
        )

    def start_gather(c, buf, sem):
        gather_desc(c, buf, sem).start()

    def wait_gather(c, buf, sem):
        gather_desc(c, buf, sem).wait()

    def start_out(c, buf, sem):
        for dt in range(D // 8):
            pltpu.async_copy(
                buf.at[pl.ds(dt * 8, 8), pl.ds(0, 128)],
                out_hbm.at[c, dt, wid],
                sem,
            )

    def wait_out(buf, sem):
        for dt in range(D // 8):
            pltpu.make_async_copy(
                buf.at[pl.ds(dt * 8, 8), pl.ds(0, 128)],
                out_hbm.at[0, dt, wid],
                sem,
            ).wait()

    lane = lax.iota(jnp.int32, 16)
    dvecs = [jnp.int32(d0) + lane for d0 in range(0, D, 16)]

    ROWS_PER_IT = 4

    def scale_t(gbuf, obuf):
        def body(it, carry):
            b0 = it * ROWS_PER_IT
            bvecs = [
                jnp.full((16,), 0, jnp.int32) + (b0 + r)
                for r in range(ROWS_PER_IT)
            ]
            vals = [
                gbuf[b0 + r, pl.ds(k * 16, 16)] * SCALE
                for r in range(ROWS_PER_IT)
                for k in range(D // 16)
            ]
            for r in range(ROWS_PER_IT):
                for k in range(D // 16):
                    plsc.store_scatter(
                        obuf,
                        [dvecs[k], bvecs[r]],
                        vals[r * (D // 16) + k],
                    )
            return carry

        lax.fori_loop(0, CHUNK // ROWS_PER_IT, body, 0)

    for b in range(NBUF - 1):
        start_gather(b, gbufs[b], gsems[b])

    def outer(p, carry):
        for b in range(NBUF):
            g = p * NBUF + b
            nb = (b + NBUF - 1) % NBUF
            nxt = g + NBUF - 1

            @pl.when(nxt < NCHUNK)
            def _issue():
                if b == 0:

                    @pl.when(p > 0)
                    def _():
                        wait_out(obufs[nb], osems[nb])

                else:
                    wait_out(obufs[nb], osems[nb])
                start_gather(nxt, gbufs[nb], gsems[nb])

            wait_gather(g, gbufs[b], gsems[b])
            scale_t(gbufs[b], obufs[b])
            start_out(g, obufs[b], osems[b])
        return carry

    lax.fori_loop(0, NOUTER, outer, 0)

    for b in range(NBUF):
        wait_out(obufs[b], osems[b])


def kernel(x, table):
    xv = jnp.swapaxes(x.astype(jnp.int32).T.reshape(LT, 8, BT, 128), 1, 2)
    out5 = _embed(xv, table)
    return out5.transpose(2, 4, 0, 1, 3).reshape(B, L, D)
